# baseline (speedup 1.0000x reference)
import functools

import jax
import jax.numpy as jnp
from jax import lax
from jax.experimental import pallas as pl
from jax.experimental.pallas import tpu as pltpu
from jax.experimental.pallas import tpu_sc as plsc

N_NODES = 10000
N_EDGES = 320000
DIM = 128
C = 128

NW = 32
NODE_FULL = N_NODES // C
NODE_TAIL = N_NODES - NODE_FULL * C
EDGE_CHUNKS = N_EDGES // C


def _body(node_idx, edge_idx, node_table, edge_table, node_out, edge_out,
          idx_buf, rows_buf, tail_idx, tail_rows, sem):
    nc = 2
    wid = lax.axis_index("s") * nc + lax.axis_index("c")

    def run_chunks(idx_hbm, table, out_hbm, count):
        def body(j, _):
            c = wid + NW * j
            base = c * C
            pltpu.sync_copy(idx_hbm.at[pl.ds(base, C)], idx_buf)
            pltpu.async_copy(table.at[idx_buf], rows_buf, sem).wait()
            pltpu.sync_copy(rows_buf, out_hbm.at[pl.ds(base, C)])
            return 0
        lax.fori_loop(0, count, body, 0)

    node_cnt = NODE_FULL // NW + jnp.where(wid < NODE_FULL % NW, 1, 0)
    run_chunks(node_idx, node_table, node_out, node_cnt)

    edge_cnt = EDGE_CHUNKS // NW + jnp.where(wid < EDGE_CHUNKS % NW, 1, 0)
    run_chunks(edge_idx, edge_table, edge_out, edge_cnt)

    @pl.when(wid == NW - 1)
    def _():
        base = NODE_FULL * C
        pltpu.sync_copy(node_idx.at[pl.ds(base, NODE_TAIL)], tail_idx)
        pltpu.async_copy(node_table.at[tail_idx], tail_rows, sem).wait()
        pltpu.sync_copy(tail_rows, node_out.at[pl.ds(base, NODE_TAIL)])


@jax.jit
def kernel(node_idx, edge_idx, node_table, edge_table):
    mesh = plsc.VectorSubcoreMesh(core_axis_name="c", subcore_axis_name="s")
    f = functools.partial(
        pl.kernel,
        out_type=(
            jax.ShapeDtypeStruct((N_NODES, DIM), jnp.float32),
            jax.ShapeDtypeStruct((N_EDGES, DIM), jnp.float32),
        ),
        mesh=mesh,
        scratch_types=[
            pltpu.VMEM((C,), jnp.int32),
            pltpu.VMEM((C, DIM), jnp.float32),
            pltpu.VMEM((NODE_TAIL,), jnp.int32),
            pltpu.VMEM((NODE_TAIL, DIM), jnp.float32),
            pltpu.SemaphoreType.DMA,
        ],
    )(_body)
    return f(node_idx, edge_idx, node_table, edge_table)


# speedup vs baseline: 1.0028x; 1.0028x over previous
import functools

import jax
import jax.numpy as jnp
from jax import lax
from jax.experimental import pallas as pl
from jax.experimental.pallas import tpu as pltpu
from jax.experimental.pallas import tpu_sc as plsc

N_NODES = 10000
N_EDGES = 320000
DIM = 128
C = 128
TAIL = 16

NW = 32
E_PER_W = N_EDGES // NW
E_FULL = E_PER_W // C
N_WORKERS_NODE = 25
N_PER_W = N_NODES // N_WORKERS_NODE
N_FULL = N_PER_W // C


def _body(node_idx, edge_idx, node_table, edge_table, node_out, edge_out,
          idx_e, idx_n, buf0, buf1, tbuf,
          gsem0, gsem1, ssem0, ssem1, tsem):
    nc = 2
    wid = lax.axis_index("s") * nc + lax.axis_index("c")

    @pl.when(wid < N_WORKERS_NODE)
    def _():
        nbase = wid * N_PER_W
        pltpu.sync_copy(node_idx.at[pl.ds(nbase, N_PER_W)], idx_n)
        for j in range(N_FULL):
            pltpu.async_copy(
                node_table.at[idx_n.at[pl.ds(j * C, C)]], buf0, gsem0).wait()
            pltpu.sync_copy(buf0, node_out.at[pl.ds(nbase + j * C, C)])
        pltpu.async_copy(
            node_table.at[idx_n.at[pl.ds(N_FULL * C, TAIL)]], tbuf, tsem).wait()
        pltpu.sync_copy(tbuf, node_out.at[pl.ds(nbase + N_FULL * C, TAIL)])

    ebase = wid * E_PER_W
    pltpu.sync_copy(edge_idx.at[pl.ds(ebase, E_PER_W)], idx_e)

    def g(j, buf, sem):
        pltpu.async_copy(edge_table.at[idx_e.at[pl.ds(j * C, C)]], buf, sem)

    def gwait(buf, sem):
        pltpu.make_async_copy(
            edge_table.at[idx_e.at[pl.ds(0, C)]], buf, sem).wait()

    def s(j, buf, sem):
        pltpu.async_copy(buf, edge_out.at[pl.ds(ebase + j * C, C)], sem)

    def swait(buf, sem):
        pltpu.make_async_copy(buf, edge_out.at[pl.ds(ebase, C)], sem).wait()

    nloop = E_FULL // 2
    g(0, buf0, gsem0)

    def body(i, _):
        c0 = 2 * i
        gwait(buf0, gsem0)
        s(c0, buf0, ssem0)

        @pl.when(i > 0)
        def _():
            swait(buf1, ssem1)
        g(c0 + 1, buf1, gsem1)
        gwait(buf1, gsem1)
        s(c0 + 1, buf1, ssem1)
        swait(buf0, ssem0)

        @pl.when(i < nloop - 1)
        def _():
            g(c0 + 2, buf0, gsem0)
        return 0

    lax.fori_loop(0, nloop, body, 0)

    pltpu.async_copy(
        edge_table.at[idx_e.at[pl.ds(E_FULL * C, TAIL)]], tbuf, tsem).wait()
    pltpu.sync_copy(tbuf, edge_out.at[pl.ds(ebase + E_FULL * C, TAIL)])
    swait(buf1, ssem1)


@jax.jit
def kernel(node_idx, edge_idx, node_table, edge_table):
    mesh = plsc.VectorSubcoreMesh(core_axis_name="c", subcore_axis_name="s")
    f = functools.partial(
        pl.kernel,
        out_type=(
            jax.ShapeDtypeStruct((N_NODES, DIM), jnp.float32),
            jax.ShapeDtypeStruct((N_EDGES, DIM), jnp.float32),
        ),
        mesh=mesh,
        scratch_types=[
            pltpu.VMEM((E_PER_W,), jnp.int32),
            pltpu.VMEM((N_PER_W,), jnp.int32),
            pltpu.VMEM((C, DIM), jnp.float32),
            pltpu.VMEM((C, DIM), jnp.float32),
            pltpu.VMEM((TAIL, DIM), jnp.float32),
            pltpu.SemaphoreType.DMA,
            pltpu.SemaphoreType.DMA,
            pltpu.SemaphoreType.DMA,
            pltpu.SemaphoreType.DMA,
            pltpu.SemaphoreType.DMA,
        ],
    )(_body)
    return f(node_idx, edge_idx, node_table, edge_table)


# speedup vs baseline: 36.0979x; 35.9979x over previous
import functools

import jax
import jax.numpy as jnp
from jax import lax
from jax.experimental import pallas as pl
from jax.experimental.pallas import tpu as pltpu
from jax.experimental.pallas import tpu_sc as plsc

N_NODES = 10000
N_EDGES = 320000
DIM = 128
C = 128
TAIL = 16

NW = 32
E_PER_W = N_EDGES // NW
E_FULL = E_PER_W // C
N_WORKERS_NODE = 25
N_PER_W = N_NODES // N_WORKERS_NODE
N_FULL = N_PER_W // C


def _body(node_idx, edge_idx, node_table, edge_table, node_out, edge_out,
          idx_e, idx_n, buf0, buf1, tbuf, ntab_sh, etab_sh,
          gsem0, gsem1, ssem0, ssem1, tsem):
    nc = 2
    sid = lax.axis_index("s")
    wid = sid * nc + lax.axis_index("c")

    @pl.when(sid == 0)
    def _():
        pltpu.sync_copy(node_table, ntab_sh)
        pltpu.sync_copy(edge_table, etab_sh)
    plsc.subcore_barrier()

    node_table = ntab_sh
    edge_table = etab_sh

    @pl.when(wid < N_WORKERS_NODE)
    def _():
        nbase = wid * N_PER_W
        pltpu.sync_copy(node_idx.at[pl.ds(nbase, N_PER_W)], idx_n)
        for j in range(N_FULL):
            pltpu.async_copy(
                node_table.at[idx_n.at[pl.ds(j * C, C)]], buf0, gsem0).wait()
            pltpu.sync_copy(buf0, node_out.at[pl.ds(nbase + j * C, C)])
        pltpu.async_copy(
            node_table.at[idx_n.at[pl.ds(N_FULL * C, TAIL)]], tbuf, tsem).wait()
        pltpu.sync_copy(tbuf, node_out.at[pl.ds(nbase + N_FULL * C, TAIL)])

    ebase = wid * E_PER_W
    pltpu.sync_copy(edge_idx.at[pl.ds(ebase, E_PER_W)], idx_e)

    def g(j, buf, sem):
        pltpu.async_copy(edge_table.at[idx_e.at[pl.ds(j * C, C)]], buf, sem)

    def gwait(buf, sem):
        pltpu.make_async_copy(
            edge_table.at[idx_e.at[pl.ds(0, C)]], buf, sem).wait()

    def s(j, buf, sem):
        pltpu.async_copy(buf, edge_out.at[pl.ds(ebase + j * C, C)], sem)

    def swait(buf, sem):
        pltpu.make_async_copy(buf, edge_out.at[pl.ds(ebase, C)], sem).wait()

    nloop = E_FULL // 2
    g(0, buf0, gsem0)

    def body(i, _):
        c0 = 2 * i
        gwait(buf0, gsem0)
        s(c0, buf0, ssem0)

        @pl.when(i > 0)
        def _():
            swait(buf1, ssem1)
        g(c0 + 1, buf1, gsem1)
        gwait(buf1, gsem1)
        s(c0 + 1, buf1, ssem1)
        swait(buf0, ssem0)

        @pl.when(i < nloop - 1)
        def _():
            g(c0 + 2, buf0, gsem0)
        return 0

    lax.fori_loop(0, nloop, body, 0)

    pltpu.async_copy(
        edge_table.at[idx_e.at[pl.ds(E_FULL * C, TAIL)]], tbuf, tsem).wait()
    pltpu.sync_copy(tbuf, edge_out.at[pl.ds(ebase + E_FULL * C, TAIL)])
    swait(buf1, ssem1)


@jax.jit
def kernel(node_idx, edge_idx, node_table, edge_table):
    mesh = plsc.VectorSubcoreMesh(core_axis_name="c", subcore_axis_name="s")
    f = functools.partial(
        pl.kernel,
        out_type=(
            jax.ShapeDtypeStruct((N_NODES, DIM), jnp.float32),
            jax.ShapeDtypeStruct((N_EDGES, DIM), jnp.float32),
        ),
        mesh=mesh,
        scratch_types=[
            pltpu.VMEM((E_PER_W,), jnp.int32),
            pltpu.VMEM((N_PER_W,), jnp.int32),
            pltpu.VMEM((C, DIM), jnp.float32),
            pltpu.VMEM((C, DIM), jnp.float32),
            pltpu.VMEM((TAIL, DIM), jnp.float32),
            pltpu.VMEM_SHARED((21, DIM), jnp.float32),
            pltpu.VMEM_SHARED((4, DIM), jnp.float32),
            pltpu.SemaphoreType.DMA,
            pltpu.SemaphoreType.DMA,
            pltpu.SemaphoreType.DMA,
            pltpu.SemaphoreType.DMA,
            pltpu.SemaphoreType.DMA,
        ],
    )(_body)
    return f(node_idx, edge_idx, node_table, edge_table)


# speedup vs baseline: 38.3267x; 1.0617x over previous
import functools

import jax
import jax.numpy as jnp
from jax import lax
from jax.experimental import pallas as pl
from jax.experimental.pallas import tpu as pltpu
from jax.experimental.pallas import tpu_sc as plsc

N_NODES = 10000
N_EDGES = 320000
DIM = 128
C = 128
TAIL = 16

NW = 32
E_PER_W = N_EDGES // NW
E_FULL = E_PER_W // C

NODE_BLK = 2000
NODE_VPAD = 32


def _edge_body(edge_idx, edge_table, edge_out,
               idx_e, buf0, buf1, tbuf, etab_sh,
               gsem0, gsem1, ssem0, ssem1, tsem):
    nc = 2
    sid = lax.axis_index("s")
    wid = sid * nc + lax.axis_index("c")

    @pl.when(sid == 0)
    def _():
        pltpu.sync_copy(edge_table, etab_sh)
    plsc.subcore_barrier()

    ebase = wid * E_PER_W
    pltpu.sync_copy(edge_idx.at[pl.ds(ebase, E_PER_W)], idx_e)

    def g(j, buf, sem):
        pltpu.async_copy(etab_sh.at[idx_e.at[pl.ds(j * C, C)]], buf, sem)

    def gwait(buf, sem):
        pltpu.make_async_copy(
            etab_sh.at[idx_e.at[pl.ds(0, C)]], buf, sem).wait()

    def s(j, buf, sem):
        pltpu.async_copy(buf, edge_out.at[pl.ds(ebase + j * C, C)], sem)

    def swait(buf, sem):
        pltpu.make_async_copy(buf, edge_out.at[pl.ds(ebase, C)], sem).wait()

    nloop = E_FULL // 2
    g(0, buf0, gsem0)

    def body(i, _):
        c0 = 2 * i
        gwait(buf0, gsem0)
        s(c0, buf0, ssem0)

        @pl.when(i > 0)
        def _():
            swait(buf1, ssem1)
        g(c0 + 1, buf1, gsem1)
        gwait(buf1, gsem1)
        s(c0 + 1, buf1, ssem1)
        swait(buf0, ssem0)

        @pl.when(i < nloop - 1)
        def _():
            g(c0 + 2, buf0, gsem0)
        return 0

    lax.fori_loop(0, nloop, body, 0)

    pltpu.async_copy(
        etab_sh.at[idx_e.at[pl.ds(E_FULL * C, TAIL)]], tbuf, tsem).wait()
    pltpu.sync_copy(tbuf, edge_out.at[pl.ds(ebase + E_FULL * C, TAIL)])
    swait(buf1, ssem1)


def _node_tc_body(idx_ref, tab_ref, out_ref):
    idx = idx_ref[0, 0, :]
    oh = (idx[:, None] == lax.broadcasted_iota(
        jnp.int32, (NODE_BLK, NODE_VPAD), 1)).astype(jnp.float32)
    out_ref[...] = jnp.dot(oh, tab_ref[...],
                           preferred_element_type=jnp.float32)


@jax.jit
def kernel(node_idx, edge_idx, node_table, edge_table):
    mesh = plsc.VectorSubcoreMesh(core_axis_name="c", subcore_axis_name="s")
    edge_fn = functools.partial(
        pl.kernel,
        out_type=jax.ShapeDtypeStruct((N_EDGES, DIM), jnp.float32),
        mesh=mesh,
        scratch_types=[
            pltpu.VMEM((E_PER_W,), jnp.int32),
            pltpu.VMEM((C, DIM), jnp.float32),
            pltpu.VMEM((C, DIM), jnp.float32),
            pltpu.VMEM((TAIL, DIM), jnp.float32),
            pltpu.VMEM_SHARED((4, DIM), jnp.float32),
            pltpu.SemaphoreType.DMA,
            pltpu.SemaphoreType.DMA,
            pltpu.SemaphoreType.DMA,
            pltpu.SemaphoreType.DMA,
            pltpu.SemaphoreType.DMA,
        ],
    )(_edge_body)
    edge_emb = edge_fn(edge_idx, edge_table)

    nb = N_NODES // NODE_BLK
    ntab = jnp.zeros((NODE_VPAD, DIM), jnp.float32).at[:21].set(node_table)
    node_emb = pl.pallas_call(
        _node_tc_body,
        grid=(nb,),
        in_specs=[
            pl.BlockSpec((1, 1, NODE_BLK), lambda i: (i, 0, 0)),
            pl.BlockSpec((NODE_VPAD, DIM), lambda i: (0, 0)),
        ],
        out_specs=pl.BlockSpec((NODE_BLK, DIM), lambda i: (i, 0)),
        out_shape=jax.ShapeDtypeStruct((N_NODES, DIM), jnp.float32),
    )(node_idx.reshape(nb, 1, NODE_BLK), ntab)

    return (node_emb, edge_emb)


# speedup vs baseline: 38.4406x; 1.0030x over previous
import functools

import jax
import jax.numpy as jnp
from jax import lax
from jax.experimental import pallas as pl
from jax.experimental.pallas import tpu as pltpu
from jax.experimental.pallas import tpu_sc as plsc

N_NODES = 10000
N_EDGES = 320000
DIM = 128
C = 128
TAIL = 16

NW = 32
E_PER_W = N_EDGES // NW
E_FULL = E_PER_W // C

NODE_BLK = 2000
NODE_VPAD = 32


def _edge_body(edge_idx, edge_table, edge_out,
               idx_e, buf0, buf1, tbuf, etab_sh,
               gsem0, gsem1, ssem0, ssem1, tsem):
    nc = 2
    sid = lax.axis_index("s")
    wid = sid * nc + lax.axis_index("c")

    @pl.when(sid == 0)
    def _():
        pltpu.sync_copy(edge_table, etab_sh)
    plsc.subcore_barrier()

    ebase = wid * E_PER_W
    pltpu.sync_copy(edge_idx.at[pl.ds(ebase, E_PER_W)], idx_e)

    def g(j, buf, sem):
        pltpu.async_copy(etab_sh.at[idx_e.at[pl.ds(j * C, C)]], buf, sem)

    def gwait(buf, sem):
        pltpu.make_async_copy(
            etab_sh.at[idx_e.at[pl.ds(0, C)]], buf, sem).wait()

    def s(j, buf, sem):
        pltpu.async_copy(buf, edge_out.at[pl.ds(ebase + j * C, C)], sem)

    def swait(buf, sem):
        pltpu.make_async_copy(buf, edge_out.at[pl.ds(ebase, C)], sem).wait()

    nloop = E_FULL // 2
    g(0, buf0, gsem0)

    def body(i, _):
        c0 = 2 * i
        gwait(buf0, gsem0)
        s(c0, buf0, ssem0)

        @pl.when(i > 0)
        def _():
            swait(buf1, ssem1)
        g(c0 + 1, buf1, gsem1)
        gwait(buf1, gsem1)
        s(c0 + 1, buf1, ssem1)
        swait(buf0, ssem0)

        @pl.when(i < nloop - 1)
        def _():
            g(c0 + 2, buf0, gsem0)
        return 0

    lax.fori_loop(0, nloop, body, 0)

    pltpu.async_copy(
        etab_sh.at[idx_e.at[pl.ds(E_FULL * C, TAIL)]], tbuf, tsem).wait()
    pltpu.sync_copy(tbuf, edge_out.at[pl.ds(ebase + E_FULL * C, TAIL)])
    swait(buf1, ssem1)


def _node_tc_body(idx_ref, tab_ref, out_ref):
    idx = idx_ref[0, 0, :]
    tab = tab_ref[...]
    idx2d = jnp.broadcast_to(idx[:, None], (NODE_BLK, DIM))
    acc = jnp.zeros((NODE_BLK, DIM), jnp.float32)
    for v in range(21):
        row = jnp.broadcast_to(tab[v][None, :], (NODE_BLK, DIM))
        acc = acc + jnp.where(idx2d == v, row, 0.0)
    out_ref[...] = acc


@jax.jit
def kernel(node_idx, edge_idx, node_table, edge_table):
    mesh = plsc.VectorSubcoreMesh(core_axis_name="c", subcore_axis_name="s")
    edge_fn = functools.partial(
        pl.kernel,
        out_type=jax.ShapeDtypeStruct((N_EDGES, DIM), jnp.float32),
        mesh=mesh,
        scratch_types=[
            pltpu.VMEM((E_PER_W,), jnp.int32),
            pltpu.VMEM((C, DIM), jnp.float32),
            pltpu.VMEM((C, DIM), jnp.float32),
            pltpu.VMEM((TAIL, DIM), jnp.float32),
            pltpu.VMEM_SHARED((4, DIM), jnp.float32),
            pltpu.SemaphoreType.DMA,
            pltpu.SemaphoreType.DMA,
            pltpu.SemaphoreType.DMA,
            pltpu.SemaphoreType.DMA,
            pltpu.SemaphoreType.DMA,
        ],
    )(_edge_body)
    edge_emb = edge_fn(edge_idx, edge_table)

    nb = N_NODES // NODE_BLK
    ntab = jnp.zeros((NODE_VPAD, DIM), jnp.float32).at[:21].set(node_table)
    node_emb = pl.pallas_call(
        _node_tc_body,
        grid=(nb,),
        in_specs=[
            pl.BlockSpec((1, 1, NODE_BLK), lambda i: (i, 0, 0)),
            pl.BlockSpec((NODE_VPAD, DIM), lambda i: (0, 0)),
        ],
        out_specs=pl.BlockSpec((NODE_BLK, DIM), lambda i: (i, 0)),
        out_shape=jax.ShapeDtypeStruct((N_NODES, DIM), jnp.float32),
    )(node_idx.reshape(nb, 1, NODE_BLK), ntab)

    return (node_emb, edge_emb)
